# baseline (speedup 1.0000x reference)
import jax
import jax.numpy as jnp
from jax.experimental import pallas as pl
from jax.experimental.pallas import tpu as pltpu

_NSEG = 64
_NEG = -1e30


def _pool_kernel(x_ref, b_ref, wg_ref, wn_ref, bn_ref,
                 out_ref, m1_ref, s1_ref, m2_ref, s2_ref, a1_ref, a2_ref):
    i = pl.program_id(0)
    nb = pl.num_programs(0)

    @pl.when(i == 0)
    def _init():
        for r in (m1_ref, m2_ref):
            r[:] = jnp.full_like(r, _NEG)
        for r in (s1_ref, s2_ref, a1_ref, a2_ref):
            r[:] = jnp.zeros_like(r)

    blk = x_ref.shape[0]
    half = blk // 2

    def _chain(lo, hi, m_ref, s_ref, acc_ref):
        xb = x_ref[lo:hi].astype(jnp.bfloat16)
        G = jnp.dot(xb, wg_ref[:], preferred_element_type=jnp.float32)
        b = jnp.transpose(b_ref[0, :, lo:hi])
        seg = jax.lax.broadcasted_iota(jnp.int32, (hi - lo, _NSEG), 1)
        mask = b == seg
        masked_G = jnp.where(mask, G, -3e38)
        bmax = jnp.max(masked_G, axis=0, keepdims=True)
        m_old = m_ref[:]
        m_new = jnp.maximum(m_old, bmax)
        alpha = jnp.exp(m_old - m_new)
        E = jnp.exp(masked_G - m_new)
        seg_e = jnp.sum(E, axis=0, keepdims=True)
        Et = jnp.transpose(E.astype(jnp.bfloat16))
        upd = jnp.dot(Et, xb, preferred_element_type=jnp.float32)
        m_ref[:] = m_new
        s_ref[:] = alpha * s_ref[:] + seg_e
        acc_ref[:] = jnp.reshape(alpha, (_NSEG, 1)) * acc_ref[:] + upd

    _chain(0, half, m1_ref, s1_ref, a1_ref)
    _chain(half, blk, m2_ref, s2_ref, a2_ref)

    @pl.when(i == nb - 1)
    def _finish():
        m1, m2 = m1_ref[:], m2_ref[:]
        m = jnp.maximum(m1, m2)
        c1 = jnp.exp(m1 - m)
        c2 = jnp.exp(m2 - m)
        s = s1_ref[:] * c1 + s2_ref[:] * c2
        acc = (a1_ref[:] * jnp.reshape(c1, (_NSEG, 1))
               + a2_ref[:] * jnp.reshape(c2, (_NSEG, 1)))
        scale = 1.0 / (s + 1e-16)
        pooled = acc * jnp.reshape(scale, (_NSEG, 1))
        out = jnp.dot(pooled, wn_ref[:], preferred_element_type=jnp.float32)
        frac = jnp.reshape(s * scale, (_NSEG, 1))
        out_ref[:] = out + frac * bn_ref[:]


def _pick_block(n):
    for blk in (5000, 2000, 1000, 500, 200, 100, 8):
        if n % blk == 0:
            return blk
    return n


def kernel(x, Wg, bg, Wn, bn, batch, size):
    n, d = x.shape
    blk = _pick_block(n)
    grid = n // blk
    b3 = batch.astype(jnp.int32).reshape(grid, 1, blk)
    wg_rep = jnp.broadcast_to(Wg, (d, _NSEG)).astype(jnp.bfloat16)
    out = pl.pallas_call(
        _pool_kernel,
        grid=(grid,),
        in_specs=[
            pl.BlockSpec((blk, d), lambda i: (i, 0)),
            pl.BlockSpec((1, 1, blk), lambda i: (i, 0, 0)),
            pl.BlockSpec((d, _NSEG), lambda i: (0, 0)),
            pl.BlockSpec((d, d), lambda i: (0, 0)),
            pl.BlockSpec((1, d), lambda i: (0, 0)),
        ],
        out_specs=pl.BlockSpec((_NSEG, d), lambda i: (0, 0)),
        out_shape=jax.ShapeDtypeStruct((_NSEG, d), jnp.float32),
        scratch_shapes=[
            pltpu.VMEM((1, _NSEG), jnp.float32),
            pltpu.VMEM((1, _NSEG), jnp.float32),
            pltpu.VMEM((1, _NSEG), jnp.float32),
            pltpu.VMEM((1, _NSEG), jnp.float32),
            pltpu.VMEM((_NSEG, d), jnp.float32),
            pltpu.VMEM((_NSEG, d), jnp.float32),
        ],
    )(x, b3, wg_rep, Wn, bn.reshape(1, d))
    return out


# speedup vs baseline: 1.3602x; 1.3602x over previous
import jax
import jax.numpy as jnp
from jax.experimental import pallas as pl
from jax.experimental.pallas import tpu as pltpu

_NSEG = 64
_NEG = -1e30


def _pool_kernel(x_ref, b_ref, wg_ref, wn_ref, bn_ref,
                 out_ref, m_ref, s_ref, acc_ref):
    i = pl.program_id(0)
    nb = pl.num_programs(0)

    @pl.when(i == 0)
    def _init():
        m_ref[:] = jnp.full_like(m_ref, _NEG)
        s_ref[:] = jnp.zeros_like(s_ref)
        acc_ref[:] = jnp.zeros_like(acc_ref)

    blk = x_ref.shape[0]
    half = blk // 2

    def _stage(lo, hi):
        xb = x_ref[lo:hi].astype(jnp.bfloat16)
        G = jnp.dot(xb, wg_ref[:], preferred_element_type=jnp.float32)
        b = jnp.transpose(b_ref[0, :, lo:hi])
        seg = jax.lax.broadcasted_iota(jnp.int32, (hi - lo, _NSEG), 1)
        mask = b == seg
        masked_G = jnp.where(mask, G, -3e38)
        bmax = jnp.max(masked_G, axis=0, keepdims=True)
        return xb, masked_G, bmax

    xb1, mG1, bmax1 = _stage(0, half)
    xb2, mG2, bmax2 = _stage(half, blk)

    m_old = m_ref[:]
    m_mid = jnp.maximum(m_old, bmax1)
    alpha1 = jnp.exp(m_old - m_mid)
    E1 = jnp.exp(mG1 - m_mid)
    seg_e1 = jnp.sum(E1, axis=0, keepdims=True)
    Et1 = jnp.transpose(E1.astype(jnp.bfloat16))
    upd1 = jnp.dot(Et1, xb1, preferred_element_type=jnp.float32)

    m_new = jnp.maximum(m_mid, bmax2)
    beta = jnp.exp(m_mid - m_new)
    E2 = jnp.exp(mG2 - m_new)
    seg_e2 = jnp.sum(E2, axis=0, keepdims=True)
    Et2 = jnp.transpose(E2.astype(jnp.bfloat16))
    upd2 = jnp.dot(Et2, xb2, preferred_element_type=jnp.float32)

    m_ref[:] = m_new
    s_ref[:] = (alpha1 * s_ref[:] + seg_e1) * beta + seg_e2
    ab = jnp.reshape(alpha1 * beta, (_NSEG, 1))
    bcol = jnp.reshape(beta, (_NSEG, 1))
    acc_ref[:] = ab * acc_ref[:] + bcol * upd1 + upd2

    @pl.when(i == nb - 1)
    def _finish():
        s = s_ref[:]
        scale = 1.0 / (s + 1e-16)
        pooled = acc_ref[:] * jnp.reshape(scale, (_NSEG, 1))
        out = jnp.dot(pooled, wn_ref[:], preferred_element_type=jnp.float32)
        frac = jnp.reshape(s * scale, (_NSEG, 1))
        out_ref[:] = out + frac * bn_ref[:]


def _pick_block(n):
    for blk in (5000, 2000, 1000, 500, 200, 100, 8):
        if n % blk == 0:
            return blk
    return n


def kernel(x, Wg, bg, Wn, bn, batch, size):
    n, d = x.shape
    blk = _pick_block(n)
    grid = n // blk
    b3 = batch.astype(jnp.int32).reshape(grid, 1, blk)
    wg_rep = jnp.broadcast_to(Wg, (d, _NSEG)).astype(jnp.bfloat16)
    out = pl.pallas_call(
        _pool_kernel,
        grid=(grid,),
        in_specs=[
            pl.BlockSpec((blk, d), lambda i: (i, 0)),
            pl.BlockSpec((1, 1, blk), lambda i: (i, 0, 0)),
            pl.BlockSpec((d, _NSEG), lambda i: (0, 0)),
            pl.BlockSpec((d, d), lambda i: (0, 0)),
            pl.BlockSpec((1, d), lambda i: (0, 0)),
        ],
        out_specs=pl.BlockSpec((_NSEG, d), lambda i: (0, 0)),
        out_shape=jax.ShapeDtypeStruct((_NSEG, d), jnp.float32),
        scratch_shapes=[
            pltpu.VMEM((1, _NSEG), jnp.float32),
            pltpu.VMEM((1, _NSEG), jnp.float32),
            pltpu.VMEM((_NSEG, d), jnp.float32),
        ],
    )(x, b3, wg_rep, Wn, bn.reshape(1, d))
    return out


# speedup vs baseline: 1.3685x; 1.0061x over previous
import jax
import jax.numpy as jnp
from jax.experimental import pallas as pl
from jax.experimental.pallas import tpu as pltpu

_NSEG = 64
_NEG = -1e30


def _pool_kernel(x_ref, b_ref, wg_ref, wn_ref, bn_ref,
                 out_ref, m_ref, s_ref, acc_ref):
    i = pl.program_id(0)
    nb = pl.num_programs(0)

    @pl.when(i == 0)
    def _init():
        m_ref[:] = jnp.full_like(m_ref, _NEG)
        s_ref[:] = jnp.zeros_like(s_ref)
        acc_ref[:] = jnp.zeros_like(acc_ref)

    blk = x_ref.shape[0]
    half = blk // 2

    def _stage(lo, hi):
        xb = x_ref[lo:hi].astype(jnp.bfloat16)
        G = jnp.dot(xb, wg_ref[:], preferred_element_type=jnp.float32)
        b = jnp.transpose(b_ref[0, :, lo:hi])
        seg = jax.lax.broadcasted_iota(jnp.int32, (hi - lo, _NSEG), 1)
        mask = b == seg
        masked_G = jnp.where(mask, G, -3e38)
        bmax = jnp.max(masked_G, axis=0, keepdims=True)
        return xb, masked_G, bmax

    xb1, mG1, bmax1 = _stage(0, half)
    xb2, mG2, bmax2 = _stage(half, blk)

    m_old = m_ref[:]
    m_new = jnp.maximum(m_old, jnp.maximum(bmax1, bmax2))
    alpha = jnp.exp(m_old - m_new)

    E1 = jnp.exp(mG1 - m_new)
    E2 = jnp.exp(mG2 - m_new)
    seg_e = (jnp.sum(E1, axis=0, keepdims=True)
             + jnp.sum(E2, axis=0, keepdims=True))
    Et1 = jnp.transpose(E1.astype(jnp.bfloat16))
    Et2 = jnp.transpose(E2.astype(jnp.bfloat16))
    acc_upd = (jnp.dot(Et1, xb1, preferred_element_type=jnp.float32)
               + jnp.dot(Et2, xb2, preferred_element_type=jnp.float32))

    m_ref[:] = m_new
    s_ref[:] = alpha * s_ref[:] + seg_e
    acc_ref[:] = jnp.reshape(alpha, (_NSEG, 1)) * acc_ref[:] + acc_upd

    @pl.when(i == nb - 1)
    def _finish():
        s = s_ref[:]
        scale = 1.0 / (s + 1e-16)
        pooled = acc_ref[:] * jnp.reshape(scale, (_NSEG, 1))
        out = jnp.dot(pooled, wn_ref[:], preferred_element_type=jnp.float32)
        frac = jnp.reshape(s * scale, (_NSEG, 1))
        out_ref[:] = out + frac * bn_ref[:]


def _pick_block(n):
    for blk in (5000, 2000, 1000, 500, 200, 100, 8):
        if n % blk == 0:
            return blk
    return n


def kernel(x, Wg, bg, Wn, bn, batch, size):
    n, d = x.shape
    blk = _pick_block(n)
    grid = n // blk
    b3 = batch.astype(jnp.int32).reshape(grid, 1, blk)
    wg_rep = jnp.broadcast_to(Wg, (d, _NSEG)).astype(jnp.bfloat16)
    out = pl.pallas_call(
        _pool_kernel,
        grid=(grid,),
        in_specs=[
            pl.BlockSpec((blk, d), lambda i: (i, 0)),
            pl.BlockSpec((1, 1, blk), lambda i: (i, 0, 0)),
            pl.BlockSpec((d, _NSEG), lambda i: (0, 0)),
            pl.BlockSpec((d, d), lambda i: (0, 0)),
            pl.BlockSpec((1, d), lambda i: (0, 0)),
        ],
        out_specs=pl.BlockSpec((_NSEG, d), lambda i: (0, 0)),
        out_shape=jax.ShapeDtypeStruct((_NSEG, d), jnp.float32),
        scratch_shapes=[
            pltpu.VMEM((1, _NSEG), jnp.float32),
            pltpu.VMEM((1, _NSEG), jnp.float32),
            pltpu.VMEM((_NSEG, d), jnp.float32),
        ],
    )(x, b3, wg_rep, Wn, bn.reshape(1, d))
    return out
